# baseline (speedup 1.0000x reference)
import functools

import jax
import jax.numpy as jnp
from jax.experimental import pallas as pl
from jax.experimental.pallas import tpu as pltpu

TOP_K = 2


def _moe_dense_kernel(num_routed, x_ref, w1_ref, b1_ref, w2_ref, b2_ref,
                      wr_ref, br_ref, out_ref):
    e = pl.program_id(1)
    xb = x_ref[...]

    logits = jnp.dot(xb, wr_ref[...], preferred_element_type=jnp.float32)
    logits = logits + br_ref[...]
    m = jnp.max(logits, axis=-1, keepdims=True)
    p = jnp.exp(logits - m)
    s = p / jnp.sum(p, axis=-1, keepdims=True)

    E = s.shape[-1]
    lane = jax.lax.broadcasted_iota(jnp.int32, s.shape, 1)
    gcol = jnp.zeros((s.shape[0], 1), dtype=jnp.float32)
    for j in range(E):
        sj = s[:, j:j + 1]
        beats = (s > sj) | ((s == sj) & (lane < j))
        rank_j = jnp.sum(beats.astype(jnp.int32), axis=-1, keepdims=True)
        g_j = jnp.where(rank_j < TOP_K, sj, 0.0)
        gcol = gcol + jnp.where(e == j, 1.0, 0.0) * g_j
    gcol = jnp.where(e >= num_routed, 1.0, gcol)

    xb16 = xb.astype(jnp.bfloat16)
    h = jnp.dot(xb16, w1_ref[0].astype(jnp.bfloat16),
                preferred_element_type=jnp.float32) + b1_ref[0]
    h = jax.nn.gelu(h)
    y = jnp.dot(h.astype(jnp.bfloat16), w2_ref[0].astype(jnp.bfloat16),
                preferred_element_type=jnp.float32) + b2_ref[0]
    contrib = gcol * y

    @pl.when(e == 0)
    def _():
        out_ref[...] = contrib

    @pl.when(e > 0)
    def _():
        out_ref[...] += contrib


def moe_dense(x, W1, b1, W2, b2, Wr, br, num_routed, tb, interpret=False):
    T, D = x.shape
    ETOT, _, H = W1.shape
    grid = (T // tb, ETOT)
    b1 = b1.reshape(ETOT, 1, H)
    b2 = b2.reshape(ETOT, 1, D)
    return pl.pallas_call(
        functools.partial(_moe_dense_kernel, num_routed),
        grid=grid,
        in_specs=[
            pl.BlockSpec((tb, D), lambda t, e: (t, 0)),
            pl.BlockSpec((1, D, H), lambda t, e: (e, 0, 0)),
            pl.BlockSpec((1, 1, H), lambda t, e: (e, 0, 0)),
            pl.BlockSpec((1, H, D), lambda t, e: (e, 0, 0)),
            pl.BlockSpec((1, 1, D), lambda t, e: (e, 0, 0)),
            pl.BlockSpec((D, br.shape[-1]), lambda t, e: (0, 0)),
            pl.BlockSpec((1, br.shape[-1]), lambda t, e: (0, 0)),
        ],
        out_specs=pl.BlockSpec((tb, D), lambda t, e: (t, 0)),
        out_shape=jax.ShapeDtypeStruct((T, D), jnp.float32),
        compiler_params=pltpu.CompilerParams(
            dimension_semantics=("parallel", "arbitrary"),
        ),
        interpret=interpret,
    )(x, W1, b1, W2, b2, Wr, br)


def kernel(x, Ws1, bs1, Ws2, bs2, We1, be1, We2, be2, Wr, br):
    T, D = x.shape
    E = We1.shape[0]
    W1 = jnp.concatenate([We1, Ws1], axis=0)
    b1 = jnp.concatenate([be1, bs1], axis=0)
    W2 = jnp.concatenate([We2, Ws2], axis=0)
    b2 = jnp.concatenate([be2, bs2], axis=0)
    br2 = br.reshape(1, -1)
    tb = min(T, 1024)
    return moe_dense(x, W1, b1, W2, b2, Wr, br2, E, tb)


# speedup vs baseline: 1.4242x; 1.4242x over previous
import functools

import jax
import jax.numpy as jnp
from jax import lax
from jax.experimental import pallas as pl
from jax.experimental.pallas import tpu as pltpu
from jax.experimental.pallas import tpu_sc as plsc

TOP_K = 2
BLK = 128
NC, NS = 2, 16


def _route_kernel(blk, xT_ref, wrT_ref, brT_ref, pcat_ref, wcol_ref, blk_ref):
    lT = jnp.dot(wrT_ref[...], xT_ref[...],
                 preferred_element_type=jnp.float32) + brT_ref[...]
    m = jnp.max(lT, axis=0, keepdims=True)
    ex = jnp.exp(lT - m)
    sT = ex / jnp.sum(ex, axis=0, keepdims=True)
    E, T = sT.shape
    row = jax.lax.broadcasted_iota(jnp.int32, sT.shape, 0)

    rows0, rows1 = [], []
    for j in range(E):
        sj = sT[j:j + 1, :]
        beats = (sT > sj) | ((sT == sj) & (row < j))
        rj = jnp.sum(beats.astype(jnp.int32), axis=0, keepdims=True)
        rows0.append((rj == 0).astype(jnp.float32))
        rows1.append((rj == 1).astype(jnp.float32))
    OH0 = jnp.concatenate(rows0, axis=0)
    OH1 = jnp.concatenate(rows1, axis=0)

    OH0i = OH0.astype(jnp.int32)
    OH1i = OH1.astype(jnp.int32)
    OHcat = jnp.concatenate([OH0i, OH1i], axis=1)
    Cinc = OHcat
    sh = 1
    while sh < 2 * T:
        z = jnp.zeros((E, sh), jnp.int32)
        Cinc = Cinc + jnp.concatenate([z, Cinc[:, :2 * T - sh]], axis=1)
        sh *= 2
    Cex = Cinc - OHcat
    tot = Cinc[:, 2 * T - 1:2 * T]
    pc = ((tot + (blk - 1)) // blk) * blk

    ss = pc
    sh = 1
    while sh < E:
        z = jnp.zeros((sh, 1), jnp.int32)
        ss = ss + jnp.concatenate([z, ss[:E - sh, :]], axis=0)
        sh *= 2
    ss = ss - pc

    p0 = jnp.sum(OH0i * (ss + Cex[:, :T]), axis=0, keepdims=True)
    p1 = jnp.sum(OH1i * (ss + Cex[:, T:]), axis=0, keepdims=True)
    pcat_ref[...] = jnp.concatenate([p0, p1], axis=0)
    w0 = jnp.sum(OH0 * sT, axis=0, keepdims=True)
    w1 = jnp.sum(OH1 * sT, axis=0, keepdims=True)
    wcol_ref[...] = jnp.concatenate([w0, w1], axis=0).T

    nblk = blk_ref.shape[-1]
    bound = ss + pc
    bstart = jax.lax.broadcasted_iota(jnp.int32, (1, nblk), 1) * blk
    be = jnp.zeros((1, nblk), jnp.int32)
    for e in range(E):
        be = be + (bstart >= bound[e:e + 1, :]).astype(jnp.int32)
    blk_ref[...] = be


def _route(xT, WrT, brT, nblk):
    E, D = WrT.shape
    T = xT.shape[1]
    return pl.pallas_call(
        functools.partial(_route_kernel, BLK),
        in_specs=[
            pl.BlockSpec((D, T), lambda: (0, 0)),
            pl.BlockSpec((E, D), lambda: (0, 0)),
            pl.BlockSpec((E, 1), lambda: (0, 0)),
        ],
        out_specs=[
            pl.BlockSpec((TOP_K, T), lambda: (0, 0)),
            pl.BlockSpec((T, TOP_K), lambda: (0, 0)),
            pl.BlockSpec((1, nblk), lambda: (0, 0)),
        ],
        out_shape=[
            jax.ShapeDtypeStruct((TOP_K, T), jnp.int32),
            jax.ShapeDtypeStruct((T, TOP_K), jnp.float32),
            jax.ShapeDtypeStruct((1, nblk), jnp.int32),
        ],
    )(xT, WrT, brT)


def _make_scatter(T, D, LPAD):
    TW = T // (NC * NS)
    mesh = plsc.VectorSubcoreMesh(core_axis_name="c", subcore_axis_name="s")

    @functools.partial(
        pl.kernel,
        out_type=jax.ShapeDtypeStruct((LPAD, D), jnp.float32),
        mesh=mesh,
        scratch_types=[
            pltpu.VMEM((TW, D), jnp.float32),
            pltpu.VMEM((TW,), jnp.int32),
            pltpu.VMEM((TW,), jnp.int32),
            pltpu.SemaphoreType.DMA,
        ],
    )
    def scat(x_hbm, pcat_hbm, xs_hbm, rows_v, idx0_v, idx1_v, sem):
        wid = lax.axis_index("s") * NC + lax.axis_index("c")
        base = wid * TW
        pltpu.sync_copy(x_hbm.at[pl.ds(base, TW)], rows_v)
        pltpu.sync_copy(pcat_hbm.at[0, pl.ds(base, TW)], idx0_v)
        pltpu.sync_copy(pcat_hbm.at[1, pl.ds(base, TW)], idx1_v)
        pltpu.async_copy(rows_v, xs_hbm.at[idx0_v], sem).wait()
        pltpu.async_copy(rows_v, xs_hbm.at[idx1_v], sem).wait()

    return scat


def _sffn_kernel(x_ref, w1_ref, b1_ref, w2_ref, b2_ref, o_ref):
    xb = x_ref[...].astype(jnp.bfloat16)
    h = jnp.dot(xb, w1_ref[...].astype(jnp.bfloat16),
                preferred_element_type=jnp.float32) + b1_ref[...]
    h = jax.nn.gelu(h)
    o_ref[...] = jnp.dot(h.astype(jnp.bfloat16), w2_ref[...].astype(jnp.bfloat16),
                         preferred_element_type=jnp.float32) + b2_ref[...]


def _shared_ffn(x, W1s, b1s, W2s, b2s, tbs):
    T, D = x.shape
    SH = W1s.shape[1]
    return pl.pallas_call(
        _sffn_kernel,
        grid=(T // tbs,),
        in_specs=[
            pl.BlockSpec((tbs, D), lambda t: (t, 0)),
            pl.BlockSpec((D, SH), lambda t: (0, 0)),
            pl.BlockSpec((1, SH), lambda t: (0, 0)),
            pl.BlockSpec((SH, D), lambda t: (0, 0)),
            pl.BlockSpec((1, D), lambda t: (0, 0)),
        ],
        out_specs=pl.BlockSpec((tbs, D), lambda t: (t, 0)),
        out_shape=jax.ShapeDtypeStruct((T, D), jnp.float32),
        compiler_params=pltpu.CompilerParams(
            dimension_semantics=("parallel",),
        ),
    )(x, W1s, b1s, W2s, b2s)


def _gffn_kernel(num_routed, be_ref, xs_ref, w1_ref, b1_ref,
                 w2_ref, b2_ref, y_ref):
    b = pl.program_id(0)

    @pl.when(be_ref[b] < num_routed)
    def _():
        xb = xs_ref[...].astype(jnp.bfloat16)
        h = jnp.dot(xb, w1_ref[0].astype(jnp.bfloat16),
                    preferred_element_type=jnp.float32) + b1_ref[0]
        h = jax.nn.gelu(h)
        y_ref[...] = jnp.dot(h.astype(jnp.bfloat16),
                             w2_ref[0].astype(jnp.bfloat16),
                             preferred_element_type=jnp.float32) + b2_ref[0]


def _grouped_ffn(be, xs, We1, be1, We2, be2):
    E, D, H = We1.shape
    LPAD = xs.shape[0]
    nblk = LPAD // BLK
    grid_spec = pltpu.PrefetchScalarGridSpec(
        num_scalar_prefetch=1,
        grid=(nblk,),
        in_specs=[
            pl.BlockSpec((BLK, D), lambda b, be_s: (b, 0)),
            pl.BlockSpec((1, D, H),
                         lambda b, be_s: (jnp.minimum(be_s[b], E - 1), 0, 0)),
            pl.BlockSpec((1, 1, H),
                         lambda b, be_s: (jnp.minimum(be_s[b], E - 1), 0, 0)),
            pl.BlockSpec((1, H, D),
                         lambda b, be_s: (jnp.minimum(be_s[b], E - 1), 0, 0)),
            pl.BlockSpec((1, 1, D),
                         lambda b, be_s: (jnp.minimum(be_s[b], E - 1), 0, 0)),
        ],
        out_specs=pl.BlockSpec((BLK, D), lambda b, be_s: (b, 0)),
    )
    return pl.pallas_call(
        functools.partial(_gffn_kernel, E),
        grid_spec=grid_spec,
        out_shape=jax.ShapeDtypeStruct((LPAD, D), jnp.float32),
        compiler_params=pltpu.CompilerParams(
            dimension_semantics=("arbitrary",),
        ),
    )(be, xs, We1, be1.reshape(E, 1, H), We2, be2.reshape(E, 1, D))


def _make_gather(T, D, LPAD):
    TW = T // (NC * NS)
    mesh = plsc.VectorSubcoreMesh(core_axis_name="c", subcore_axis_name="s")

    @functools.partial(
        pl.kernel,
        out_type=[
            jax.ShapeDtypeStruct((T, D), jnp.float32),
            jax.ShapeDtypeStruct((T, D), jnp.float32),
        ],
        mesh=mesh,
        scratch_types=[
            pltpu.VMEM((TW, D), jnp.float32),
            pltpu.VMEM((TW,), jnp.int32),
            pltpu.SemaphoreType.DMA,
        ],
    )
    def gath(y_hbm, pcat_hbm, y0_hbm, y1_hbm, rows_v, idx_v, sem):
        wid = lax.axis_index("s") * NC + lax.axis_index("c")
        base = wid * TW
        pltpu.sync_copy(pcat_hbm.at[0, pl.ds(base, TW)], idx_v)
        pltpu.async_copy(y_hbm.at[idx_v], rows_v, sem).wait()
        pltpu.sync_copy(rows_v, y0_hbm.at[pl.ds(base, TW)])
        pltpu.sync_copy(pcat_hbm.at[1, pl.ds(base, TW)], idx_v)
        pltpu.async_copy(y_hbm.at[idx_v], rows_v, sem).wait()
        pltpu.sync_copy(rows_v, y1_hbm.at[pl.ds(base, TW)])

    return gath


def _add_kernel(a_ref, b_ref, c_ref, w_ref, o_ref):
    w = w_ref[...]
    o_ref[...] = (a_ref[...] + w[:, 0:1] * b_ref[...]
                  + w[:, 1:2] * c_ref[...])


def _add3(a, b, c, wcol, tbs):
    T, D = a.shape
    return pl.pallas_call(
        _add_kernel,
        grid=(T // tbs,),
        in_specs=[pl.BlockSpec((tbs, D), lambda t: (t, 0))] * 3
        + [pl.BlockSpec((tbs, TOP_K), lambda t: (t, 0))],
        out_specs=pl.BlockSpec((tbs, D), lambda t: (t, 0)),
        out_shape=jax.ShapeDtypeStruct((T, D), jnp.float32),
        compiler_params=pltpu.CompilerParams(
            dimension_semantics=("parallel",),
        ),
    )(a, b, c, wcol)


def kernel(x, Ws1, bs1, Ws2, bs2, We1, be1, We2, be2, Wr, br):
    T, D = x.shape
    E, _, H = We1.shape
    S = Ws1.shape[0]
    nblk = (T * TOP_K) // BLK + E
    LPAD = nblk * BLK

    pcat, wcol, blk_map = _route(x.T, Wr.T, br.reshape(E, 1), nblk)

    xs = _make_scatter(T, D, LPAD)(x, pcat)

    W1s = jnp.concatenate([Ws1[s] for s in range(S)], axis=1)
    b1s = jnp.concatenate([bs1[s] for s in range(S)], axis=0).reshape(1, -1)
    W2s = jnp.concatenate([Ws2[s] for s in range(S)], axis=0)
    b2s = jnp.sum(bs2, axis=0).reshape(1, D)
    shared = _shared_ffn(x, W1s, b1s, W2s, b2s, tbs=512)

    y = _grouped_ffn(blk_map.reshape(nblk), xs, We1, be1, We2, be2)

    y0, y1 = _make_gather(T, D, LPAD)(y, pcat)
    return _add3(shared, y0, y1, wcol, tbs=512)


# speedup vs baseline: 1.4756x; 1.0361x over previous
import functools

import jax
import jax.numpy as jnp
from jax import lax
from jax.experimental import pallas as pl
from jax.experimental.pallas import tpu as pltpu
from jax.experimental.pallas import tpu_sc as plsc

TOP_K = 2
BLK = 128
NC, NS = 2, 16


def _route_kernel(blk, x_ref, wr_ref, br_ref, pcat_ref, wcol_ref, blk_ref):
    logits = jnp.dot(x_ref[...], wr_ref[...],
                     preferred_element_type=jnp.float32) + br_ref[...]
    lT = logits.T
    m = jnp.max(lT, axis=0, keepdims=True)
    ex = jnp.exp(lT - m)
    sT = ex / jnp.sum(ex, axis=0, keepdims=True)
    E, T = sT.shape
    row = jax.lax.broadcasted_iota(jnp.int32, sT.shape, 0)

    rows0, rows1 = [], []
    for j in range(E):
        sj = sT[j:j + 1, :]
        beats = (sT > sj) | ((sT == sj) & (row < j))
        rj = jnp.sum(beats.astype(jnp.int32), axis=0, keepdims=True)
        rows0.append((rj == 0).astype(jnp.float32))
        rows1.append((rj == 1).astype(jnp.float32))
    OH0 = jnp.concatenate(rows0, axis=0)
    OH1 = jnp.concatenate(rows1, axis=0)

    OH0i = OH0.astype(jnp.int32)
    OH1i = OH1.astype(jnp.int32)
    OHcat = jnp.concatenate([OH0i, OH1i], axis=1)
    Cinc = OHcat
    sh = 1
    while sh < 2 * T:
        z = jnp.zeros((E, sh), jnp.int32)
        Cinc = Cinc + jnp.concatenate([z, Cinc[:, :2 * T - sh]], axis=1)
        sh *= 2
    Cex = Cinc - OHcat
    tot = Cinc[:, 2 * T - 1:2 * T]
    pc = ((tot + (blk - 1)) // blk) * blk

    ss = pc
    sh = 1
    while sh < E:
        z = jnp.zeros((sh, 1), jnp.int32)
        ss = ss + jnp.concatenate([z, ss[:E - sh, :]], axis=0)
        sh *= 2
    ss = ss - pc

    p0 = jnp.sum(OH0i * (ss + Cex[:, :T]), axis=0, keepdims=True)
    p1 = jnp.sum(OH1i * (ss + Cex[:, T:]), axis=0, keepdims=True)
    pcat_ref[...] = jnp.concatenate([p0, p1], axis=0)
    w0 = jnp.sum(OH0 * sT, axis=0, keepdims=True)
    w1 = jnp.sum(OH1 * sT, axis=0, keepdims=True)
    wcol_ref[...] = jnp.concatenate([w0, w1], axis=0).T

    nblk = blk_ref.shape[-1]
    bound = ss + pc
    bstart = jax.lax.broadcasted_iota(jnp.int32, (1, nblk), 1) * blk
    be = jnp.zeros((1, nblk), jnp.int32)
    for e in range(E):
        be = be + (bstart >= bound[e:e + 1, :]).astype(jnp.int32)
    blk_ref[...] = be


def _route(x, Wr, br2, nblk):
    D, E = Wr.shape
    T = x.shape[0]
    return pl.pallas_call(
        functools.partial(_route_kernel, BLK),
        in_specs=[
            pl.BlockSpec((T, D), lambda: (0, 0)),
            pl.BlockSpec((D, E), lambda: (0, 0)),
            pl.BlockSpec((1, E), lambda: (0, 0)),
        ],
        out_specs=[
            pl.BlockSpec((TOP_K, T), lambda: (0, 0)),
            pl.BlockSpec((T, TOP_K), lambda: (0, 0)),
            pl.BlockSpec((1, nblk), lambda: (0, 0)),
        ],
        out_shape=[
            jax.ShapeDtypeStruct((TOP_K, T), jnp.int32),
            jax.ShapeDtypeStruct((T, TOP_K), jnp.float32),
            jax.ShapeDtypeStruct((1, nblk), jnp.int32),
        ],
    )(x, Wr, br2)


def _make_scatter(T, D, LPAD):
    TW = T // (NC * NS)
    mesh = plsc.VectorSubcoreMesh(core_axis_name="c", subcore_axis_name="s")

    @functools.partial(
        pl.kernel,
        out_type=jax.ShapeDtypeStruct((LPAD, D), jnp.float32),
        mesh=mesh,
        scratch_types=[
            pltpu.VMEM((TW, D), jnp.float32),
            pltpu.VMEM((TW,), jnp.int32),
            pltpu.VMEM((TW,), jnp.int32),
            pltpu.SemaphoreType.DMA,
        ],
    )
    def scat(x_hbm, pcat_hbm, xs_hbm, rows_v, idx0_v, idx1_v, sem):
        wid = lax.axis_index("s") * NC + lax.axis_index("c")
        base = wid * TW
        pltpu.sync_copy(x_hbm.at[pl.ds(base, TW)], rows_v)
        pltpu.sync_copy(pcat_hbm.at[0, pl.ds(base, TW)], idx0_v)
        pltpu.sync_copy(pcat_hbm.at[1, pl.ds(base, TW)], idx1_v)
        pltpu.async_copy(rows_v, xs_hbm.at[idx0_v], sem).wait()
        pltpu.async_copy(rows_v, xs_hbm.at[idx1_v], sem).wait()

    return scat


def _sffn_kernel(x_ref, w1_ref, b1_ref, w2_ref, b2_ref, o_ref):
    xb = x_ref[...].astype(jnp.bfloat16)
    h = jnp.dot(xb, w1_ref[...].astype(jnp.bfloat16),
                preferred_element_type=jnp.float32) + b1_ref[...]
    h = jax.nn.gelu(h)
    o_ref[...] = jnp.dot(h.astype(jnp.bfloat16), w2_ref[...].astype(jnp.bfloat16),
                         preferred_element_type=jnp.float32) + b2_ref[...]


def _shared_ffn(x, W1s, b1s, W2s, b2s, tbs):
    T, D = x.shape
    SH = W1s.shape[1]
    return pl.pallas_call(
        _sffn_kernel,
        grid=(T // tbs,),
        in_specs=[
            pl.BlockSpec((tbs, D), lambda t: (t, 0)),
            pl.BlockSpec((D, SH), lambda t: (0, 0)),
            pl.BlockSpec((1, SH), lambda t: (0, 0)),
            pl.BlockSpec((SH, D), lambda t: (0, 0)),
            pl.BlockSpec((1, D), lambda t: (0, 0)),
        ],
        out_specs=pl.BlockSpec((tbs, D), lambda t: (t, 0)),
        out_shape=jax.ShapeDtypeStruct((T, D), jnp.float32),
        compiler_params=pltpu.CompilerParams(
            dimension_semantics=("parallel",),
        ),
    )(x, W1s, b1s, W2s, b2s)


def _gffn_kernel(num_routed, be_ref, xs_ref, w1_ref, b1_ref,
                 w2_ref, b2_ref, y_ref):
    b = pl.program_id(0)

    @pl.when(be_ref[b] < num_routed)
    def _():
        xb = xs_ref[...].astype(jnp.bfloat16)
        h = jnp.dot(xb, w1_ref[0].astype(jnp.bfloat16),
                    preferred_element_type=jnp.float32) + b1_ref[0]
        h = jax.nn.gelu(h)
        y_ref[...] = jnp.dot(h.astype(jnp.bfloat16),
                             w2_ref[0].astype(jnp.bfloat16),
                             preferred_element_type=jnp.float32) + b2_ref[0]


def _grouped_ffn(be, xs, We1, be1, We2, be2):
    E, D, H = We1.shape
    LPAD = xs.shape[0]
    nblk = LPAD // BLK
    grid_spec = pltpu.PrefetchScalarGridSpec(
        num_scalar_prefetch=1,
        grid=(nblk,),
        in_specs=[
            pl.BlockSpec((BLK, D), lambda b, be_s: (b, 0)),
            pl.BlockSpec((1, D, H),
                         lambda b, be_s: (jnp.minimum(be_s[b], E - 1), 0, 0)),
            pl.BlockSpec((1, 1, H),
                         lambda b, be_s: (jnp.minimum(be_s[b], E - 1), 0, 0)),
            pl.BlockSpec((1, H, D),
                         lambda b, be_s: (jnp.minimum(be_s[b], E - 1), 0, 0)),
            pl.BlockSpec((1, 1, D),
                         lambda b, be_s: (jnp.minimum(be_s[b], E - 1), 0, 0)),
        ],
        out_specs=pl.BlockSpec((BLK, D), lambda b, be_s: (b, 0)),
    )
    return pl.pallas_call(
        functools.partial(_gffn_kernel, E),
        grid_spec=grid_spec,
        out_shape=jax.ShapeDtypeStruct((LPAD, D), jnp.float32),
        compiler_params=pltpu.CompilerParams(
            dimension_semantics=("arbitrary",),
        ),
    )(be, xs, We1, be1.reshape(E, 1, H), We2, be2.reshape(E, 1, D))


def _make_gather(T, D, LPAD):
    TW = T // (NC * NS)
    mesh = plsc.VectorSubcoreMesh(core_axis_name="c", subcore_axis_name="s")

    @functools.partial(
        pl.kernel,
        out_type=[
            jax.ShapeDtypeStruct((T, D), jnp.float32),
            jax.ShapeDtypeStruct((T, D), jnp.float32),
        ],
        mesh=mesh,
        scratch_types=[
            pltpu.VMEM((TW, D), jnp.float32),
            pltpu.VMEM((TW,), jnp.int32),
            pltpu.SemaphoreType.DMA,
        ],
    )
    def gath(y_hbm, pcat_hbm, y0_hbm, y1_hbm, rows_v, idx_v, sem):
        wid = lax.axis_index("s") * NC + lax.axis_index("c")
        base = wid * TW
        pltpu.sync_copy(pcat_hbm.at[0, pl.ds(base, TW)], idx_v)
        pltpu.async_copy(y_hbm.at[idx_v], rows_v, sem).wait()
        pltpu.sync_copy(rows_v, y0_hbm.at[pl.ds(base, TW)])
        pltpu.sync_copy(pcat_hbm.at[1, pl.ds(base, TW)], idx_v)
        pltpu.async_copy(y_hbm.at[idx_v], rows_v, sem).wait()
        pltpu.sync_copy(rows_v, y1_hbm.at[pl.ds(base, TW)])

    return gath


def _add_kernel(a_ref, b_ref, c_ref, w_ref, o_ref):
    w = w_ref[...]
    o_ref[...] = (a_ref[...] + w[:, 0:1] * b_ref[...]
                  + w[:, 1:2] * c_ref[...])


def _add3(a, b, c, wcol, tbs):
    T, D = a.shape
    return pl.pallas_call(
        _add_kernel,
        grid=(T // tbs,),
        in_specs=[pl.BlockSpec((tbs, D), lambda t: (t, 0))] * 3
        + [pl.BlockSpec((tbs, TOP_K), lambda t: (t, 0))],
        out_specs=pl.BlockSpec((tbs, D), lambda t: (t, 0)),
        out_shape=jax.ShapeDtypeStruct((T, D), jnp.float32),
        compiler_params=pltpu.CompilerParams(
            dimension_semantics=("parallel",),
        ),
    )(a, b, c, wcol)


def kernel(x, Ws1, bs1, Ws2, bs2, We1, be1, We2, be2, Wr, br):
    T, D = x.shape
    E, _, H = We1.shape
    S = Ws1.shape[0]
    nblk = (T * TOP_K) // BLK + E
    LPAD = nblk * BLK

    pcat, wcol, blk_map = _route(x, Wr, br.reshape(1, E), nblk)

    xs = _make_scatter(T, D, LPAD)(x, pcat)

    W1s = jnp.concatenate([Ws1[s] for s in range(S)], axis=1)
    b1s = jnp.concatenate([bs1[s] for s in range(S)], axis=0).reshape(1, -1)
    W2s = jnp.concatenate([Ws2[s] for s in range(S)], axis=0)
    b2s = jnp.sum(bs2, axis=0).reshape(1, D)
    shared = _shared_ffn(x, W1s, b1s, W2s, b2s, tbs=512)

    y = _grouped_ffn(blk_map.reshape(nblk), xs, We1, be1, We2, be2)

    y0, y1 = _make_gather(T, D, LPAD)(y, pcat)
    return _add3(shared, y0, y1, wcol, tbs=512)


# speedup vs baseline: 1.5347x; 1.0401x over previous
import functools

import jax
import jax.numpy as jnp
from jax import lax
from jax.experimental import pallas as pl
from jax.experimental.pallas import tpu as pltpu
from jax.experimental.pallas import tpu_sc as plsc

TOP_K = 2
BLK = 256
NC, NS = 2, 16


def _route_kernel(blk, x_ref, wr_ref, br_ref, pcat_ref, wcol_ref, blk_ref):
    logits = jnp.dot(x_ref[...], wr_ref[...],
                     preferred_element_type=jnp.float32) + br_ref[...]
    lT = logits.T
    m = jnp.max(lT, axis=0, keepdims=True)
    ex = jnp.exp(lT - m)
    sT = ex / jnp.sum(ex, axis=0, keepdims=True)
    E, T = sT.shape
    row = jax.lax.broadcasted_iota(jnp.int32, sT.shape, 0)

    rows0, rows1 = [], []
    for j in range(E):
        sj = sT[j:j + 1, :]
        beats = (sT > sj) | ((sT == sj) & (row < j))
        rj = jnp.sum(beats.astype(jnp.int32), axis=0, keepdims=True)
        rows0.append((rj == 0).astype(jnp.float32))
        rows1.append((rj == 1).astype(jnp.float32))
    OH0 = jnp.concatenate(rows0, axis=0)
    OH1 = jnp.concatenate(rows1, axis=0)

    OH0i = OH0.astype(jnp.int32)
    OH1i = OH1.astype(jnp.int32)
    OHcat = jnp.concatenate([OH0i, OH1i], axis=1)
    Cinc = OHcat
    sh = 1
    while sh < 2 * T:
        z = jnp.zeros((E, sh), jnp.int32)
        Cinc = Cinc + jnp.concatenate([z, Cinc[:, :2 * T - sh]], axis=1)
        sh *= 2
    Cex = Cinc - OHcat
    tot = Cinc[:, 2 * T - 1:2 * T]
    pc = ((tot + (blk - 1)) // blk) * blk

    ss = pc
    sh = 1
    while sh < E:
        z = jnp.zeros((sh, 1), jnp.int32)
        ss = ss + jnp.concatenate([z, ss[:E - sh, :]], axis=0)
        sh *= 2
    ss = ss - pc

    p0 = jnp.sum(OH0i * (ss + Cex[:, :T]), axis=0, keepdims=True)
    p1 = jnp.sum(OH1i * (ss + Cex[:, T:]), axis=0, keepdims=True)
    pcat_ref[...] = jnp.concatenate([p0, p1], axis=0)
    w0 = jnp.sum(OH0 * sT, axis=0, keepdims=True)
    w1 = jnp.sum(OH1 * sT, axis=0, keepdims=True)
    wcol_ref[...] = jnp.concatenate([w0, w1], axis=0).T

    nblk = blk_ref.shape[-1]
    bound = ss + pc
    bstart = jax.lax.broadcasted_iota(jnp.int32, (1, nblk), 1) * blk
    be = jnp.zeros((1, nblk), jnp.int32)
    for e in range(E):
        be = be + (bstart >= bound[e:e + 1, :]).astype(jnp.int32)
    blk_ref[...] = be


def _route(x, Wr, br2, nblk):
    D, E = Wr.shape
    T = x.shape[0]
    return pl.pallas_call(
        functools.partial(_route_kernel, BLK),
        in_specs=[
            pl.BlockSpec((T, D), lambda: (0, 0)),
            pl.BlockSpec((D, E), lambda: (0, 0)),
            pl.BlockSpec((1, E), lambda: (0, 0)),
        ],
        out_specs=[
            pl.BlockSpec((TOP_K, T), lambda: (0, 0)),
            pl.BlockSpec((T, TOP_K), lambda: (0, 0)),
            pl.BlockSpec((1, nblk), lambda: (0, 0)),
        ],
        out_shape=[
            jax.ShapeDtypeStruct((TOP_K, T), jnp.int32),
            jax.ShapeDtypeStruct((T, TOP_K), jnp.float32),
            jax.ShapeDtypeStruct((1, nblk), jnp.int32),
        ],
    )(x, Wr, br2)


def _make_scatter(T, D, LPAD):
    TW = T // (NC * NS)
    mesh = plsc.VectorSubcoreMesh(core_axis_name="c", subcore_axis_name="s")

    @functools.partial(
        pl.kernel,
        out_type=jax.ShapeDtypeStruct((LPAD, D), jnp.float32),
        mesh=mesh,
        scratch_types=[
            pltpu.VMEM((TW, D), jnp.float32),
            pltpu.VMEM((TW,), jnp.int32),
            pltpu.VMEM((TW,), jnp.int32),
            pltpu.SemaphoreType.DMA,
        ],
    )
    def scat(x_hbm, pcat_hbm, xs_hbm, rows_v, idx0_v, idx1_v, sem):
        wid = lax.axis_index("s") * NC + lax.axis_index("c")
        base = wid * TW
        pltpu.sync_copy(x_hbm.at[pl.ds(base, TW)], rows_v)
        pltpu.sync_copy(pcat_hbm.at[0, pl.ds(base, TW)], idx0_v)
        pltpu.sync_copy(pcat_hbm.at[1, pl.ds(base, TW)], idx1_v)
        pltpu.async_copy(rows_v, xs_hbm.at[idx0_v], sem).wait()
        pltpu.async_copy(rows_v, xs_hbm.at[idx1_v], sem).wait()

    return scat


def _sffn_kernel(x_ref, w1_ref, b1_ref, w2_ref, b2_ref, o_ref):
    xb = x_ref[...].astype(jnp.bfloat16)
    h = jnp.dot(xb, w1_ref[...].astype(jnp.bfloat16),
                preferred_element_type=jnp.float32) + b1_ref[...]
    h = jax.nn.gelu(h)
    o_ref[...] = jnp.dot(h.astype(jnp.bfloat16), w2_ref[...].astype(jnp.bfloat16),
                         preferred_element_type=jnp.float32) + b2_ref[...]


def _shared_ffn(x, W1s, b1s, W2s, b2s, tbs):
    T, D = x.shape
    SH = W1s.shape[1]
    return pl.pallas_call(
        _sffn_kernel,
        grid=(T // tbs,),
        in_specs=[
            pl.BlockSpec((tbs, D), lambda t: (t, 0)),
            pl.BlockSpec((D, SH), lambda t: (0, 0)),
            pl.BlockSpec((1, SH), lambda t: (0, 0)),
            pl.BlockSpec((SH, D), lambda t: (0, 0)),
            pl.BlockSpec((1, D), lambda t: (0, 0)),
        ],
        out_specs=pl.BlockSpec((tbs, D), lambda t: (t, 0)),
        out_shape=jax.ShapeDtypeStruct((T, D), jnp.float32),
        compiler_params=pltpu.CompilerParams(
            dimension_semantics=("parallel",),
        ),
    )(x, W1s, b1s, W2s, b2s)


def _gffn_kernel(num_routed, be_ref, xs_ref, w1_ref, b1_ref,
                 w2_ref, b2_ref, y_ref):
    b = pl.program_id(0)

    @pl.when(be_ref[b] < num_routed)
    def _():
        xb = xs_ref[...].astype(jnp.bfloat16)
        h = jnp.dot(xb, w1_ref[0].astype(jnp.bfloat16),
                    preferred_element_type=jnp.float32) + b1_ref[0]
        h = jax.nn.gelu(h)
        y_ref[...] = jnp.dot(h.astype(jnp.bfloat16),
                             w2_ref[0].astype(jnp.bfloat16),
                             preferred_element_type=jnp.float32) + b2_ref[0]


def _grouped_ffn(be, xs, We1, be1, We2, be2):
    E, D, H = We1.shape
    LPAD = xs.shape[0]
    nblk = LPAD // BLK
    grid_spec = pltpu.PrefetchScalarGridSpec(
        num_scalar_prefetch=1,
        grid=(nblk,),
        in_specs=[
            pl.BlockSpec((BLK, D), lambda b, be_s: (b, 0)),
            pl.BlockSpec((1, D, H),
                         lambda b, be_s: (jnp.minimum(be_s[b], E - 1), 0, 0)),
            pl.BlockSpec((1, 1, H),
                         lambda b, be_s: (jnp.minimum(be_s[b], E - 1), 0, 0)),
            pl.BlockSpec((1, H, D),
                         lambda b, be_s: (jnp.minimum(be_s[b], E - 1), 0, 0)),
            pl.BlockSpec((1, 1, D),
                         lambda b, be_s: (jnp.minimum(be_s[b], E - 1), 0, 0)),
        ],
        out_specs=pl.BlockSpec((BLK, D), lambda b, be_s: (b, 0)),
    )
    return pl.pallas_call(
        functools.partial(_gffn_kernel, E),
        grid_spec=grid_spec,
        out_shape=jax.ShapeDtypeStruct((LPAD, D), jnp.float32),
        compiler_params=pltpu.CompilerParams(
            dimension_semantics=("arbitrary",),
        ),
    )(be, xs, We1, be1.reshape(E, 1, H), We2, be2.reshape(E, 1, D))


def _make_gather(T, D, LPAD):
    TW = T // (NC * NS)
    mesh = plsc.VectorSubcoreMesh(core_axis_name="c", subcore_axis_name="s")

    @functools.partial(
        pl.kernel,
        out_type=[
            jax.ShapeDtypeStruct((T, D), jnp.float32),
            jax.ShapeDtypeStruct((T, D), jnp.float32),
        ],
        mesh=mesh,
        scratch_types=[
            pltpu.VMEM((TW, D), jnp.float32),
            pltpu.VMEM((TW,), jnp.int32),
            pltpu.SemaphoreType.DMA,
        ],
    )
    def gath(y_hbm, pcat_hbm, y0_hbm, y1_hbm, rows_v, idx_v, sem):
        wid = lax.axis_index("s") * NC + lax.axis_index("c")
        base = wid * TW
        pltpu.sync_copy(pcat_hbm.at[0, pl.ds(base, TW)], idx_v)
        pltpu.async_copy(y_hbm.at[idx_v], rows_v, sem).wait()
        pltpu.sync_copy(rows_v, y0_hbm.at[pl.ds(base, TW)])
        pltpu.sync_copy(pcat_hbm.at[1, pl.ds(base, TW)], idx_v)
        pltpu.async_copy(y_hbm.at[idx_v], rows_v, sem).wait()
        pltpu.sync_copy(rows_v, y1_hbm.at[pl.ds(base, TW)])

    return gath


def _add_kernel(a_ref, b_ref, c_ref, w_ref, o_ref):
    w = w_ref[...]
    o_ref[...] = (a_ref[...] + w[:, 0:1] * b_ref[...]
                  + w[:, 1:2] * c_ref[...])


def _add3(a, b, c, wcol, tbs):
    T, D = a.shape
    return pl.pallas_call(
        _add_kernel,
        grid=(T // tbs,),
        in_specs=[pl.BlockSpec((tbs, D), lambda t: (t, 0))] * 3
        + [pl.BlockSpec((tbs, TOP_K), lambda t: (t, 0))],
        out_specs=pl.BlockSpec((tbs, D), lambda t: (t, 0)),
        out_shape=jax.ShapeDtypeStruct((T, D), jnp.float32),
        compiler_params=pltpu.CompilerParams(
            dimension_semantics=("parallel",),
        ),
    )(a, b, c, wcol)


def kernel(x, Ws1, bs1, Ws2, bs2, We1, be1, We2, be2, Wr, br):
    T, D = x.shape
    E, _, H = We1.shape
    S = Ws1.shape[0]
    nblk = (T * TOP_K) // BLK + E
    LPAD = nblk * BLK

    pcat, wcol, blk_map = _route(x, Wr, br.reshape(1, E), nblk)

    xs = _make_scatter(T, D, LPAD)(x, pcat)

    W1s = jnp.concatenate([Ws1[s] for s in range(S)], axis=1)
    b1s = jnp.concatenate([bs1[s] for s in range(S)], axis=0).reshape(1, -1)
    W2s = jnp.concatenate([Ws2[s] for s in range(S)], axis=0)
    b2s = jnp.sum(bs2, axis=0).reshape(1, D)
    shared = _shared_ffn(x, W1s, b1s, W2s, b2s, tbs=512)

    y = _grouped_ffn(blk_map.reshape(nblk), xs, We1, be1, We2, be2)

    y0, y1 = _make_gather(T, D, LPAD)(y, pcat)
    return _add3(shared, y0, y1, wcol, tbs=512)


# speedup vs baseline: 1.5828x; 1.0313x over previous
import functools

import jax
import jax.numpy as jnp
from jax import lax
from jax.experimental import pallas as pl
from jax.experimental.pallas import tpu as pltpu
from jax.experimental.pallas import tpu_sc as plsc

TOP_K = 2
BLK = 512
NC, NS = 2, 16


def _route_kernel(blk, x_ref, wr_ref, br_ref, pcat_ref, wcol_ref, blk_ref):
    logits = jnp.dot(x_ref[...], wr_ref[...],
                     preferred_element_type=jnp.float32) + br_ref[...]
    lT = logits.T
    m = jnp.max(lT, axis=0, keepdims=True)
    ex = jnp.exp(lT - m)
    sT = ex / jnp.sum(ex, axis=0, keepdims=True)
    E, T = sT.shape
    row = jax.lax.broadcasted_iota(jnp.int32, sT.shape, 0)

    rows0, rows1 = [], []
    for j in range(E):
        sj = sT[j:j + 1, :]
        beats = (sT > sj) | ((sT == sj) & (row < j))
        rj = jnp.sum(beats.astype(jnp.int32), axis=0, keepdims=True)
        rows0.append((rj == 0).astype(jnp.float32))
        rows1.append((rj == 1).astype(jnp.float32))
    OH0 = jnp.concatenate(rows0, axis=0)
    OH1 = jnp.concatenate(rows1, axis=0)

    OH0i = OH0.astype(jnp.int32)
    OH1i = OH1.astype(jnp.int32)
    OHcat = jnp.concatenate([OH0i, OH1i], axis=1)
    Cinc = OHcat
    sh = 1
    while sh < 2 * T:
        z = jnp.zeros((E, sh), jnp.int32)
        Cinc = Cinc + jnp.concatenate([z, Cinc[:, :2 * T - sh]], axis=1)
        sh *= 2
    Cex = Cinc - OHcat
    tot = Cinc[:, 2 * T - 1:2 * T]
    pc = ((tot + (blk - 1)) // blk) * blk

    ss = pc
    sh = 1
    while sh < E:
        z = jnp.zeros((sh, 1), jnp.int32)
        ss = ss + jnp.concatenate([z, ss[:E - sh, :]], axis=0)
        sh *= 2
    ss = ss - pc

    p0 = jnp.sum(OH0i * (ss + Cex[:, :T]), axis=0, keepdims=True)
    p1 = jnp.sum(OH1i * (ss + Cex[:, T:]), axis=0, keepdims=True)
    pcat_ref[...] = jnp.concatenate([p0, p1], axis=0)
    w0 = jnp.sum(OH0 * sT, axis=0, keepdims=True)
    w1 = jnp.sum(OH1 * sT, axis=0, keepdims=True)
    wcol_ref[...] = jnp.concatenate([w0, w1], axis=0).T

    nblk = blk_ref.shape[-1]
    bound = ss + pc
    bstart = jax.lax.broadcasted_iota(jnp.int32, (1, nblk), 1) * blk
    be = jnp.zeros((1, nblk), jnp.int32)
    for e in range(E):
        be = be + (bstart >= bound[e:e + 1, :]).astype(jnp.int32)
    blk_ref[...] = be


def _route(x, Wr, br2, nblk):
    D, E = Wr.shape
    T = x.shape[0]
    return pl.pallas_call(
        functools.partial(_route_kernel, BLK),
        in_specs=[
            pl.BlockSpec((T, D), lambda: (0, 0)),
            pl.BlockSpec((D, E), lambda: (0, 0)),
            pl.BlockSpec((1, E), lambda: (0, 0)),
        ],
        out_specs=[
            pl.BlockSpec((TOP_K, T), lambda: (0, 0)),
            pl.BlockSpec((T, TOP_K), lambda: (0, 0)),
            pl.BlockSpec((1, nblk), lambda: (0, 0)),
        ],
        out_shape=[
            jax.ShapeDtypeStruct((TOP_K, T), jnp.int32),
            jax.ShapeDtypeStruct((T, TOP_K), jnp.float32),
            jax.ShapeDtypeStruct((1, nblk), jnp.int32),
        ],
    )(x, Wr, br2)


def _make_scatter(T, D, LPAD):
    TW = T // (NC * NS)
    mesh = plsc.VectorSubcoreMesh(core_axis_name="c", subcore_axis_name="s")

    @functools.partial(
        pl.kernel,
        out_type=jax.ShapeDtypeStruct((LPAD, D), jnp.float32),
        mesh=mesh,
        scratch_types=[
            pltpu.VMEM((TW, D), jnp.float32),
            pltpu.VMEM((TW,), jnp.int32),
            pltpu.VMEM((TW,), jnp.int32),
            pltpu.SemaphoreType.DMA,
        ],
    )
    def scat(x_hbm, pcat_hbm, xs_hbm, rows_v, idx0_v, idx1_v, sem):
        wid = lax.axis_index("s") * NC + lax.axis_index("c")
        base = wid * TW
        pltpu.sync_copy(x_hbm.at[pl.ds(base, TW)], rows_v)
        pltpu.sync_copy(pcat_hbm.at[0, pl.ds(base, TW)], idx0_v)
        pltpu.sync_copy(pcat_hbm.at[1, pl.ds(base, TW)], idx1_v)
        pltpu.async_copy(rows_v, xs_hbm.at[idx0_v], sem).wait()
        pltpu.async_copy(rows_v, xs_hbm.at[idx1_v], sem).wait()

    return scat


def _sffn_kernel(x_ref, w1_ref, b1_ref, w2_ref, b2_ref, o_ref):
    xb = x_ref[...].astype(jnp.bfloat16)
    h = jnp.dot(xb, w1_ref[...].astype(jnp.bfloat16),
                preferred_element_type=jnp.float32) + b1_ref[...]
    h = jax.nn.gelu(h)
    o_ref[...] = jnp.dot(h.astype(jnp.bfloat16), w2_ref[...].astype(jnp.bfloat16),
                         preferred_element_type=jnp.float32) + b2_ref[...]


def _shared_ffn(x, W1s, b1s, W2s, b2s, tbs):
    T, D = x.shape
    SH = W1s.shape[1]
    return pl.pallas_call(
        _sffn_kernel,
        grid=(T // tbs,),
        in_specs=[
            pl.BlockSpec((tbs, D), lambda t: (t, 0)),
            pl.BlockSpec((D, SH), lambda t: (0, 0)),
            pl.BlockSpec((1, SH), lambda t: (0, 0)),
            pl.BlockSpec((SH, D), lambda t: (0, 0)),
            pl.BlockSpec((1, D), lambda t: (0, 0)),
        ],
        out_specs=pl.BlockSpec((tbs, D), lambda t: (t, 0)),
        out_shape=jax.ShapeDtypeStruct((T, D), jnp.float32),
        compiler_params=pltpu.CompilerParams(
            dimension_semantics=("parallel",),
        ),
    )(x, W1s, b1s, W2s, b2s)


def _gffn_kernel(num_routed, be_ref, xs_ref, w1_ref, b1_ref,
                 w2_ref, b2_ref, y_ref):
    b = pl.program_id(0)

    @pl.when(be_ref[b] < num_routed)
    def _():
        xb = xs_ref[...].astype(jnp.bfloat16)
        h = jnp.dot(xb, w1_ref[0].astype(jnp.bfloat16),
                    preferred_element_type=jnp.float32) + b1_ref[0]
        h = jax.nn.gelu(h)
        y_ref[...] = jnp.dot(h.astype(jnp.bfloat16),
                             w2_ref[0].astype(jnp.bfloat16),
                             preferred_element_type=jnp.float32) + b2_ref[0]


def _grouped_ffn(be, xs, We1, be1, We2, be2):
    E, D, H = We1.shape
    LPAD = xs.shape[0]
    nblk = LPAD // BLK
    grid_spec = pltpu.PrefetchScalarGridSpec(
        num_scalar_prefetch=1,
        grid=(nblk,),
        in_specs=[
            pl.BlockSpec((BLK, D), lambda b, be_s: (b, 0)),
            pl.BlockSpec((1, D, H),
                         lambda b, be_s: (jnp.minimum(be_s[b], E - 1), 0, 0)),
            pl.BlockSpec((1, 1, H),
                         lambda b, be_s: (jnp.minimum(be_s[b], E - 1), 0, 0)),
            pl.BlockSpec((1, H, D),
                         lambda b, be_s: (jnp.minimum(be_s[b], E - 1), 0, 0)),
            pl.BlockSpec((1, 1, D),
                         lambda b, be_s: (jnp.minimum(be_s[b], E - 1), 0, 0)),
        ],
        out_specs=pl.BlockSpec((BLK, D), lambda b, be_s: (b, 0)),
    )
    return pl.pallas_call(
        functools.partial(_gffn_kernel, E),
        grid_spec=grid_spec,
        out_shape=jax.ShapeDtypeStruct((LPAD, D), jnp.float32),
        compiler_params=pltpu.CompilerParams(
            dimension_semantics=("arbitrary",),
        ),
    )(be, xs, We1, be1.reshape(E, 1, H), We2, be2.reshape(E, 1, D))


def _make_gather(T, D, LPAD):
    TW = T // (NC * NS)
    mesh = plsc.VectorSubcoreMesh(core_axis_name="c", subcore_axis_name="s")

    @functools.partial(
        pl.kernel,
        out_type=[
            jax.ShapeDtypeStruct((T, D), jnp.float32),
            jax.ShapeDtypeStruct((T, D), jnp.float32),
        ],
        mesh=mesh,
        scratch_types=[
            pltpu.VMEM((TW, D), jnp.float32),
            pltpu.VMEM((TW,), jnp.int32),
            pltpu.SemaphoreType.DMA,
        ],
    )
    def gath(y_hbm, pcat_hbm, y0_hbm, y1_hbm, rows_v, idx_v, sem):
        wid = lax.axis_index("s") * NC + lax.axis_index("c")
        base = wid * TW
        pltpu.sync_copy(pcat_hbm.at[0, pl.ds(base, TW)], idx_v)
        pltpu.async_copy(y_hbm.at[idx_v], rows_v, sem).wait()
        pltpu.sync_copy(rows_v, y0_hbm.at[pl.ds(base, TW)])
        pltpu.sync_copy(pcat_hbm.at[1, pl.ds(base, TW)], idx_v)
        pltpu.async_copy(y_hbm.at[idx_v], rows_v, sem).wait()
        pltpu.sync_copy(rows_v, y1_hbm.at[pl.ds(base, TW)])

    return gath


def _add_kernel(a_ref, b_ref, c_ref, w_ref, o_ref):
    w = w_ref[...]
    o_ref[...] = (a_ref[...] + w[:, 0:1] * b_ref[...]
                  + w[:, 1:2] * c_ref[...])


def _add3(a, b, c, wcol, tbs):
    T, D = a.shape
    return pl.pallas_call(
        _add_kernel,
        grid=(T // tbs,),
        in_specs=[pl.BlockSpec((tbs, D), lambda t: (t, 0))] * 3
        + [pl.BlockSpec((tbs, TOP_K), lambda t: (t, 0))],
        out_specs=pl.BlockSpec((tbs, D), lambda t: (t, 0)),
        out_shape=jax.ShapeDtypeStruct((T, D), jnp.float32),
        compiler_params=pltpu.CompilerParams(
            dimension_semantics=("parallel",),
        ),
    )(a, b, c, wcol)


def kernel(x, Ws1, bs1, Ws2, bs2, We1, be1, We2, be2, Wr, br):
    T, D = x.shape
    E, _, H = We1.shape
    S = Ws1.shape[0]
    nblk = (T * TOP_K) // BLK + E
    LPAD = nblk * BLK

    pcat, wcol, blk_map = _route(x, Wr, br.reshape(1, E), nblk)

    xs = _make_scatter(T, D, LPAD)(x, pcat)

    W1s = jnp.concatenate([Ws1[s] for s in range(S)], axis=1)
    b1s = jnp.concatenate([bs1[s] for s in range(S)], axis=0).reshape(1, -1)
    W2s = jnp.concatenate([Ws2[s] for s in range(S)], axis=0)
    b2s = jnp.sum(bs2, axis=0).reshape(1, D)
    shared = _shared_ffn(x, W1s, b1s, W2s, b2s, tbs=512)

    y = _grouped_ffn(blk_map.reshape(nblk), xs, We1, be1, We2, be2)

    y0, y1 = _make_gather(T, D, LPAD)(y, pcat)
    return _add3(shared, y0, y1, wcol, tbs=512)


# speedup vs baseline: 1.6536x; 1.0448x over previous
import functools

import jax
import jax.numpy as jnp
from jax import lax
from jax.experimental import pallas as pl
from jax.experimental.pallas import tpu as pltpu
from jax.experimental.pallas import tpu_sc as plsc

TOP_K = 2
BLK = 1024
SUB = 128
NC, NS = 2, 16


def _route_kernel(blk, x_ref, wr_ref, br_ref, pcat_ref, wcol_ref, blk_ref):
    logits = jnp.dot(x_ref[...], wr_ref[...],
                     preferred_element_type=jnp.float32) + br_ref[...]
    lT = logits.T
    m = jnp.max(lT, axis=0, keepdims=True)
    ex = jnp.exp(lT - m)
    sT = ex / jnp.sum(ex, axis=0, keepdims=True)
    E, T = sT.shape
    row = jax.lax.broadcasted_iota(jnp.int32, sT.shape, 0)

    rows0, rows1 = [], []
    for j in range(E):
        sj = sT[j:j + 1, :]
        beats = (sT > sj) | ((sT == sj) & (row < j))
        rj = jnp.sum(beats.astype(jnp.int32), axis=0, keepdims=True)
        rows0.append((rj == 0).astype(jnp.float32))
        rows1.append((rj == 1).astype(jnp.float32))
    OH0 = jnp.concatenate(rows0, axis=0)
    OH1 = jnp.concatenate(rows1, axis=0)

    OH0i = OH0.astype(jnp.int32)
    OH1i = OH1.astype(jnp.int32)
    OHcat = jnp.concatenate([OH0i, OH1i], axis=1)
    Cinc = OHcat
    sh = 1
    while sh < 2 * T:
        z = jnp.zeros((E, sh), jnp.int32)
        Cinc = Cinc + jnp.concatenate([z, Cinc[:, :2 * T - sh]], axis=1)
        sh *= 2
    Cex = Cinc - OHcat
    tot = Cinc[:, 2 * T - 1:2 * T]
    pc = ((tot + (blk - 1)) // blk) * blk

    ss = pc
    sh = 1
    while sh < E:
        z = jnp.zeros((sh, 1), jnp.int32)
        ss = ss + jnp.concatenate([z, ss[:E - sh, :]], axis=0)
        sh *= 2
    ss = ss - pc

    p0 = jnp.sum(OH0i * (ss + Cex[:, :T]), axis=0, keepdims=True)
    p1 = jnp.sum(OH1i * (ss + Cex[:, T:]), axis=0, keepdims=True)
    pcat_ref[...] = jnp.concatenate([p0, p1], axis=0)
    w0 = jnp.sum(OH0 * sT, axis=0, keepdims=True)
    w1 = jnp.sum(OH1 * sT, axis=0, keepdims=True)
    wcol_ref[...] = jnp.concatenate([w0, w1], axis=0).T

    nblk = blk_ref.shape[-1]
    bound = ss + pc
    bstart = jax.lax.broadcasted_iota(jnp.int32, (1, nblk), 1) * blk
    be = jnp.zeros((1, nblk), jnp.int32)
    fill = jnp.zeros((1, nblk), jnp.int32)
    for e in range(E):
        be = be + (bstart >= bound[e:e + 1, :]).astype(jnp.int32)
        inseg = (bstart >= ss[e:e + 1, :]) & (bstart < bound[e:e + 1, :])
        real_end = ss[e:e + 1, :] + tot[e:e + 1, :]
        fe = jnp.clip(real_end - bstart, 0, blk)
        fill = fill + jnp.where(inseg, fe, 0)
    blk_ref[...] = jnp.concatenate([be, fill], axis=0)


def _route(x, Wr, br2, nblk):
    D, E = Wr.shape
    T = x.shape[0]
    return pl.pallas_call(
        functools.partial(_route_kernel, BLK),
        in_specs=[
            pl.BlockSpec((T, D), lambda: (0, 0)),
            pl.BlockSpec((D, E), lambda: (0, 0)),
            pl.BlockSpec((1, E), lambda: (0, 0)),
        ],
        out_specs=[
            pl.BlockSpec((TOP_K, T), lambda: (0, 0)),
            pl.BlockSpec((T, TOP_K), lambda: (0, 0)),
            pl.BlockSpec((2, nblk), lambda: (0, 0)),
        ],
        out_shape=[
            jax.ShapeDtypeStruct((TOP_K, T), jnp.int32),
            jax.ShapeDtypeStruct((T, TOP_K), jnp.float32),
            jax.ShapeDtypeStruct((2, nblk), jnp.int32),
        ],
    )(x, Wr, br2)


def _make_scatter(T, D, LPAD):
    TW = T // (NC * NS)
    mesh = plsc.VectorSubcoreMesh(core_axis_name="c", subcore_axis_name="s")

    @functools.partial(
        pl.kernel,
        out_type=jax.ShapeDtypeStruct((LPAD, D), jnp.float32),
        mesh=mesh,
        scratch_types=[
            pltpu.VMEM((TW, D), jnp.float32),
            pltpu.VMEM((TW,), jnp.int32),
            pltpu.VMEM((TW,), jnp.int32),
            pltpu.SemaphoreType.DMA,
        ],
    )
    def scat(x_hbm, pcat_hbm, xs_hbm, rows_v, idx0_v, idx1_v, sem):
        wid = lax.axis_index("s") * NC + lax.axis_index("c")
        base = wid * TW
        pltpu.sync_copy(x_hbm.at[pl.ds(base, TW)], rows_v)
        pltpu.sync_copy(pcat_hbm.at[0, pl.ds(base, TW)], idx0_v)
        pltpu.sync_copy(pcat_hbm.at[1, pl.ds(base, TW)], idx1_v)
        pltpu.async_copy(rows_v, xs_hbm.at[idx0_v], sem).wait()
        pltpu.async_copy(rows_v, xs_hbm.at[idx1_v], sem).wait()

    return scat


def _sffn_kernel(x_ref, w1_ref, b1_ref, w2_ref, b2_ref, o_ref):
    xb = x_ref[...].astype(jnp.bfloat16)
    h = jnp.dot(xb, w1_ref[...].astype(jnp.bfloat16),
                preferred_element_type=jnp.float32) + b1_ref[...]
    h = jax.nn.gelu(h)
    o_ref[...] = jnp.dot(h.astype(jnp.bfloat16), w2_ref[...].astype(jnp.bfloat16),
                         preferred_element_type=jnp.float32) + b2_ref[...]


def _shared_ffn(x, W1s, b1s, W2s, b2s, tbs):
    T, D = x.shape
    SH = W1s.shape[1]
    return pl.pallas_call(
        _sffn_kernel,
        grid=(T // tbs,),
        in_specs=[
            pl.BlockSpec((tbs, D), lambda t: (t, 0)),
            pl.BlockSpec((D, SH), lambda t: (0, 0)),
            pl.BlockSpec((1, SH), lambda t: (0, 0)),
            pl.BlockSpec((SH, D), lambda t: (0, 0)),
            pl.BlockSpec((1, D), lambda t: (0, 0)),
        ],
        out_specs=pl.BlockSpec((tbs, D), lambda t: (t, 0)),
        out_shape=jax.ShapeDtypeStruct((T, D), jnp.float32),
        compiler_params=pltpu.CompilerParams(
            dimension_semantics=("parallel",),
        ),
    )(x, W1s, b1s, W2s, b2s)


def _gffn_kernel(nblk, be_ref, xs_ref, w1_ref, b1_ref,
                 w2_ref, b2_ref, y_ref):
    b = pl.program_id(0)
    fill = be_ref[nblk + b]
    w1 = w1_ref[0].astype(jnp.bfloat16)
    w2 = w2_ref[0].astype(jnp.bfloat16)
    for sub in range(BLK // SUB):
        @pl.when(fill > sub * SUB)
        def _():
            sl = pl.ds(sub * SUB, SUB)
            xb = xs_ref[sl, :].astype(jnp.bfloat16)
            h = jnp.dot(xb, w1, preferred_element_type=jnp.float32) + b1_ref[0]
            h = jax.nn.gelu(h)
            y_ref[sl, :] = jnp.dot(h.astype(jnp.bfloat16), w2,
                                   preferred_element_type=jnp.float32) + b2_ref[0]


def _grouped_ffn(be_fill, xs, We1, be1, We2, be2):
    E, D, H = We1.shape
    LPAD = xs.shape[0]
    nblk = LPAD // BLK
    grid_spec = pltpu.PrefetchScalarGridSpec(
        num_scalar_prefetch=1,
        grid=(nblk,),
        in_specs=[
            pl.BlockSpec((BLK, D), lambda b, be_s: (b, 0)),
            pl.BlockSpec((1, D, H),
                         lambda b, be_s: (jnp.minimum(be_s[b], E - 1), 0, 0)),
            pl.BlockSpec((1, 1, H),
                         lambda b, be_s: (jnp.minimum(be_s[b], E - 1), 0, 0)),
            pl.BlockSpec((1, H, D),
                         lambda b, be_s: (jnp.minimum(be_s[b], E - 1), 0, 0)),
            pl.BlockSpec((1, 1, D),
                         lambda b, be_s: (jnp.minimum(be_s[b], E - 1), 0, 0)),
        ],
        out_specs=pl.BlockSpec((BLK, D), lambda b, be_s: (b, 0)),
    )
    return pl.pallas_call(
        functools.partial(_gffn_kernel, nblk),
        grid_spec=grid_spec,
        out_shape=jax.ShapeDtypeStruct((LPAD, D), jnp.float32),
        compiler_params=pltpu.CompilerParams(
            dimension_semantics=("arbitrary",),
        ),
    )(be_fill, xs, We1, be1.reshape(E, 1, H), We2, be2.reshape(E, 1, D))


def _make_gather(T, D, LPAD):
    TW = T // (NC * NS)
    mesh = plsc.VectorSubcoreMesh(core_axis_name="c", subcore_axis_name="s")

    @functools.partial(
        pl.kernel,
        out_type=[
            jax.ShapeDtypeStruct((T, D), jnp.float32),
            jax.ShapeDtypeStruct((T, D), jnp.float32),
        ],
        mesh=mesh,
        scratch_types=[
            pltpu.VMEM((TW, D), jnp.float32),
            pltpu.VMEM((TW,), jnp.int32),
            pltpu.SemaphoreType.DMA,
        ],
    )
    def gath(y_hbm, pcat_hbm, y0_hbm, y1_hbm, rows_v, idx_v, sem):
        wid = lax.axis_index("s") * NC + lax.axis_index("c")
        base = wid * TW
        pltpu.sync_copy(pcat_hbm.at[0, pl.ds(base, TW)], idx_v)
        pltpu.async_copy(y_hbm.at[idx_v], rows_v, sem).wait()
        pltpu.sync_copy(rows_v, y0_hbm.at[pl.ds(base, TW)])
        pltpu.sync_copy(pcat_hbm.at[1, pl.ds(base, TW)], idx_v)
        pltpu.async_copy(y_hbm.at[idx_v], rows_v, sem).wait()
        pltpu.sync_copy(rows_v, y1_hbm.at[pl.ds(base, TW)])

    return gath


def _add_kernel(a_ref, b_ref, c_ref, w_ref, o_ref):
    w = w_ref[...]
    o_ref[...] = (a_ref[...] + w[:, 0:1] * b_ref[...]
                  + w[:, 1:2] * c_ref[...])


def _add3(a, b, c, wcol, tbs):
    T, D = a.shape
    return pl.pallas_call(
        _add_kernel,
        grid=(T // tbs,),
        in_specs=[pl.BlockSpec((tbs, D), lambda t: (t, 0))] * 3
        + [pl.BlockSpec((tbs, TOP_K), lambda t: (t, 0))],
        out_specs=pl.BlockSpec((tbs, D), lambda t: (t, 0)),
        out_shape=jax.ShapeDtypeStruct((T, D), jnp.float32),
        compiler_params=pltpu.CompilerParams(
            dimension_semantics=("parallel",),
        ),
    )(a, b, c, wcol)


def kernel(x, Ws1, bs1, Ws2, bs2, We1, be1, We2, be2, Wr, br):
    T, D = x.shape
    E, _, H = We1.shape
    S = Ws1.shape[0]
    nblk = (T * TOP_K) // BLK + E
    LPAD = nblk * BLK

    pcat, wcol, blk_map = _route(x, Wr, br.reshape(1, E), nblk)

    xs = _make_scatter(T, D, LPAD)(x, pcat)

    W1s = jnp.concatenate([Ws1[s] for s in range(S)], axis=1)
    b1s = jnp.concatenate([bs1[s] for s in range(S)], axis=0).reshape(1, -1)
    W2s = jnp.concatenate([Ws2[s] for s in range(S)], axis=0)
    b2s = jnp.sum(bs2, axis=0).reshape(1, D)
    shared = _shared_ffn(x, W1s, b1s, W2s, b2s, tbs=512)

    y = _grouped_ffn(blk_map.reshape(2 * nblk), xs, We1, be1, We2, be2)

    y0, y1 = _make_gather(T, D, LPAD)(y, pcat)
    return _add3(shared, y0, y1, wcol, tbs=512)
